# Initial kernel scaffold; baseline (speedup 1.0000x reference)
import jax
import jax.numpy as jnp
from jax.experimental import pallas as pl


def kernel(x):
    raise NotImplementedError("write your pallas kernel here")


# speedup vs baseline: 1.3251x; 1.3251x over previous
import functools

import jax
import jax.numpy as jnp
from jax import lax
from jax.experimental import pallas as pl
from jax.experimental.pallas import tpu as pltpu
from jax.experimental.pallas import tpu_sc as plsc

B, S, D = 4, 4096, 2048
ROWS = B * S
NC, NS = 2, 16
NW = NC * NS
ROWS_W = ROWS // NW
GROUP = 8
NGROUP = ROWS_W // GROUP
LANES = 16
NV = D // LANES

_mesh = plsc.VectorSubcoreMesh(core_axis_name="c", subcore_axis_name="s")


def _last(v):
    return lax.squeeze(lax.slice(v, (LANES - 1,), (LANES,)), (0,))


@functools.partial(
    pl.kernel,
    mesh=_mesh,
    out_type=jax.ShapeDtypeStruct((ROWS, D), jnp.float32),
    scratch_types=[
        pltpu.VMEM((GROUP, D), jnp.float32),
    ],
    compiler_params=pltpu.CompilerParams(needs_layout_passes=False),
)
def _cumsum_rows(x_hbm, out_hbm, buf):
    wid = lax.axis_index("s") * NC + lax.axis_index("c")
    base = wid * ROWS_W

    def group_body(g, carry):
        row0 = base + g * GROUP
        pltpu.sync_copy(x_hbm.at[pl.ds(row0, GROUP)], buf)

        def step(i, carries):
            off = i * LANES
            new = []
            for r in range(GROUP):
                v = buf[r, pl.ds(off, LANES)]
                s = plsc.cumsum(v) + carries[r]
                buf[r, pl.ds(off, LANES)] = s
                new.append(_last(s))
            return tuple(new)

        lax.fori_loop(0, NV, step, tuple(jnp.float32(0.0) for _ in range(GROUP)))
        pltpu.sync_copy(buf, out_hbm.at[pl.ds(row0, GROUP)])
        return carry

    lax.fori_loop(0, NGROUP, group_body, 0)


def kernel(x):
    out = _cumsum_rows(x.reshape(ROWS, D))
    return out.reshape(B, S, D)
